# Initial kernel scaffold; baseline (speedup 1.0000x reference)
import functools

import jax
import jax.numpy as jnp
from jax import lax
from jax.experimental import pallas as pl
from jax.experimental.pallas import tpu as pltpu

N = 100000
D = 128
G = 64
BLK = 2000
NBLK = N // BLK


def _fused_body(nodes_ref, gid_ref, w_ref, b_ref, out_ref, acc_ref):
    i = pl.program_id(0)

    x = nodes_ref[...]
    pre = lax.dot_general(x, w_ref[...], (((1,), (1,)), ((), ())),
                          preferred_element_type=jnp.float32)
    pre = pre + b_ref[...]
    act = jax.nn.selu(pre)

    gid = gid_ref[0, 0, :]
    cols = lax.broadcasted_iota(jnp.int32, (BLK, G), 1)
    onehot = (gid[:, None] == cols).astype(jnp.float32)
    part = lax.dot_general(onehot, act, (((0,), (0,)), ((), ())),
                           preferred_element_type=jnp.float32)

    @pl.when(i == 0)
    def _init():
        acc_ref[...] = jnp.zeros_like(acc_ref)

    acc_ref[...] += part

    @pl.when(i == NBLK - 1)
    def _fin():
        out_ref[...] = jnp.tanh(acc_ref[...])


@jax.jit
def kernel(nodes, graph_id, W, b):
    gid3 = graph_id.reshape(NBLK, 1, BLK)
    out = pl.pallas_call(
        _fused_body,
        grid=(NBLK,),
        in_specs=[
            pl.BlockSpec((BLK, D), lambda i: (i, 0)),
            pl.BlockSpec((1, 1, BLK), lambda i: (i, 0, 0)),
            pl.BlockSpec((D, D), lambda i: (0, 0)),
            pl.BlockSpec((1, D), lambda i: (0, 0)),
        ],
        out_specs=pl.BlockSpec((G, D), lambda i: (0, 0)),
        out_shape=jax.ShapeDtypeStruct((G, D), jnp.float32),
        scratch_shapes=[pltpu.VMEM((G, D), jnp.float32)],
    )(nodes, gid3, W, b.reshape(1, D))
    return out


# speedup vs baseline: 7.2761x; 7.2761x over previous
import functools

import jax
import jax.numpy as jnp
from jax import lax
from jax.experimental import pallas as pl
from jax.experimental.pallas import tpu as pltpu

N = 100000
D = 128
G = 64
BLK = 2000
NBLK = N // BLK


def _fused_body(nodes_ref, gid_ref, w_ref, b_ref, out_ref, acc_ref):
    i = pl.program_id(0)

    x = nodes_ref[...]
    pre = lax.dot_general(x, w_ref[...], (((1,), (1,)), ((), ())),
                          preferred_element_type=jnp.float32)
    pre = pre + b_ref[...]
    scale = 1.0507009873554804934193349852946
    alpha = 1.6732632423543772848170429916717
    neg = alpha * (jnp.exp(jnp.minimum(pre, 0.0)) - 1.0)
    act = scale * jnp.where(pre > 0, pre, neg)

    gid = gid_ref[0, 0, :]
    cols = lax.broadcasted_iota(jnp.int32, (BLK, G), 1)
    onehot = (gid[:, None] == cols).astype(jnp.float32)
    part = lax.dot_general(onehot, act, (((0,), (0,)), ((), ())),
                           preferred_element_type=jnp.float32)

    @pl.when(i == 0)
    def _init():
        acc_ref[...] = jnp.zeros_like(acc_ref)

    acc_ref[...] += part

    @pl.when(i == NBLK - 1)
    def _fin():
        out_ref[...] = jnp.tanh(acc_ref[...])


@jax.jit
def kernel(nodes, graph_id, W, b):
    gid3 = graph_id.reshape(NBLK, 1, BLK)
    out = pl.pallas_call(
        _fused_body,
        grid=(NBLK,),
        in_specs=[
            pl.BlockSpec((BLK, D), lambda i: (i, 0)),
            pl.BlockSpec((1, 1, BLK), lambda i: (i, 0, 0)),
            pl.BlockSpec((D, D), lambda i: (0, 0)),
            pl.BlockSpec((1, D), lambda i: (0, 0)),
        ],
        out_specs=pl.BlockSpec((G, D), lambda i: (0, 0)),
        out_shape=jax.ShapeDtypeStruct((G, D), jnp.float32),
        scratch_shapes=[pltpu.VMEM((G, D), jnp.float32)],
    )(nodes, gid3, W, b.reshape(1, D))
    return out
